# baseline (speedup 1.0000x reference)
import jax
import jax.numpy as jnp
from jax.experimental import pallas as pl
from jax.experimental.pallas import tpu as pltpu


def _broadcast_body(param_ref, out0_ref, out1_ref):
    out0_ref[...] = jnp.full(out0_ref.shape, param_ref[0], jnp.float32)
    out1_ref[...] = jnp.full(out1_ref.shape, param_ref[1], jnp.float32)


def kernel(x, x_pa, param):
    B = x.shape[0]
    rows = B // 128
    out0, out1 = pl.pallas_call(
        _broadcast_body,
        grid_spec=pltpu.PrefetchScalarGridSpec(num_scalar_prefetch=1),
        out_shape=(
            jax.ShapeDtypeStruct((rows, 128), jnp.float32),
            jax.ShapeDtypeStruct((rows, 128), jnp.float32),
        ),
    )(param)
    return (out0.reshape(B, 1), out1.reshape(B, 1))


# speedup vs baseline: 1.0032x; 1.0032x over previous
import jax
import jax.numpy as jnp
from jax.experimental import pallas as pl
from jax.experimental.pallas import tpu as pltpu


def _broadcast_body(param_ref, out0_ref, out1_ref):
    out0_ref[...] = jnp.full(out0_ref.shape, param_ref[0], jnp.float32)
    out1_ref[...] = jnp.full(out1_ref.shape, param_ref[1], jnp.float32)


def kernel(x, x_pa, param):
    B = x.shape[0]
    rows = B // 128
    out0, out1 = pl.pallas_call(
        _broadcast_body,
        in_specs=[pl.BlockSpec(memory_space=pltpu.SMEM)],
        out_shape=(
            jax.ShapeDtypeStruct((rows, 128), jnp.float32),
            jax.ShapeDtypeStruct((rows, 128), jnp.float32),
        ),
    )(param)
    return (out0.reshape(B, 1), out1.reshape(B, 1))
